# baseline (speedup 1.0000x reference)
import functools

import jax
import jax.numpy as jnp
from jax import lax
from jax.experimental import pallas as pl
from jax.experimental.pallas import tpu as pltpu
from jax.experimental.pallas import tpu_sc as plsc

NC = 2
NS = 16
L = 16
NW = NC * NS

B = 16384
D = 64
FIELD0 = 1_000_000

BPW = B // NW
CHUNK = 128
NCHUNK = BPW // CHUNK


def _ncf_body(x_hbm, table_hbm, wb_hbm, out_hbm,
              x_v, idx_u, idx_i, u_rows, i_rows, wb_v, tile_v, out_v, sem):
    wid = lax.axis_index("s") * NC + lax.axis_index("c")
    base = wid * BPW

    iota = lax.broadcasted_iota(jnp.int32, (L,), 0)
    zeros = jnp.zeros((L,), jnp.int32)
    ones = jnp.ones((L,), jnp.int32)

    pltpu.sync_copy(wb_hbm, wb_v)
    pltpu.sync_copy(x_hbm.at[pl.ds(2 * base, 2 * BPW)], x_v)

    for g in range(BPW // L):
        flat = (g * L + iota) * 2
        u_ix = plsc.load_gather(x_v, [flat])
        i_ix = plsc.load_gather(x_v, [flat + 1]) + FIELD0
        c, o = g // (CHUNK // L), (g % (CHUNK // L)) * L
        idx_u[c, pl.ds(o, L)] = u_ix
        idx_i[c, pl.ds(o, L)] = i_ix

    copies = []
    for c in range(NCHUNK):
        copies.append(pltpu.async_copy(table_hbm.at[idx_u.at[c]], u_rows.at[c], sem))
        copies.append(pltpu.async_copy(table_hbm.at[idx_i.at[c]], i_rows.at[c], sem))
    for cp in copies:
        cp.wait()

    w0 = wb_v[pl.ds(0, L)]
    w1 = wb_v[pl.ds(L, L)]
    w2 = wb_v[pl.ds(2 * L, L)]
    w3 = wb_v[pl.ds(3 * L, L)]
    bias = wb_v[pl.ds(D, L)]

    def group_body(g, _):
        c = g // (CHUNK // L)
        e0 = (g % (CHUNK // L)) * L
        for j in range(L):
            e = e0 + j
            p = (u_rows[c, e, pl.ds(0, L)] * i_rows[c, e, pl.ds(0, L)] * w0
                 + u_rows[c, e, pl.ds(L, L)] * i_rows[c, e, pl.ds(L, L)] * w1
                 + u_rows[c, e, pl.ds(2 * L, L)] * i_rows[c, e, pl.ds(2 * L, L)] * w2
                 + u_rows[c, e, pl.ds(3 * L, L)] * i_rows[c, e, pl.ds(3 * L, L)] * w3)
            plsc.store_scatter(tile_v, [iota * L + j], p)
        acc = bias
        for r in range(L):
            acc = acc + tile_v[pl.ds(r * L, L)]
        sig = 1.0 / (1.0 + jnp.exp(-acc))
        out_v[pl.ds(g * L, L)] = sig
        return 0

    lax.fori_loop(0, BPW // L, group_body, 0)

    pltpu.sync_copy(out_v, out_hbm.at[pl.ds(base, BPW)])


@jax.jit
def _ncf(x2, table, wb):
    mesh = plsc.VectorSubcoreMesh(
        core_axis_name="c", subcore_axis_name="s", num_cores=NC, num_subcores=NS
    )
    return pl.kernel(
        _ncf_body,
        out_type=jax.ShapeDtypeStruct((B,), jnp.float32),
        mesh=mesh,
        compiler_params=pltpu.CompilerParams(
            needs_layout_passes=False, use_tc_tiling_on_sc=False
        ),
        scratch_types=[
            pltpu.VMEM((2 * BPW,), jnp.int32),
            pltpu.VMEM((NCHUNK, CHUNK), jnp.int32),
            pltpu.VMEM((NCHUNK, CHUNK), jnp.int32),
            pltpu.VMEM((NCHUNK, CHUNK, D), jnp.float32),
            pltpu.VMEM((NCHUNK, CHUNK, D), jnp.float32),
            pltpu.VMEM((80,), jnp.float32),
            pltpu.VMEM((L * L,), jnp.float32),
            pltpu.VMEM((BPW,), jnp.float32),
            pltpu.SemaphoreType.DMA,
        ],
    )(x2, table, wb)


def kernel(x, table, W_fc, b_fc):
    x2 = x.astype(jnp.int32).reshape(2 * B)
    wb = jnp.concatenate(
        [W_fc.reshape(D), jnp.broadcast_to(b_fc.astype(jnp.float32), (16,))]
    )
    out = _ncf(x2, table, wb)
    return (out, x)
